# baseline (speedup 1.0000x reference)
import functools

import jax
import jax.numpy as jnp
from jax import lax
from jax.experimental import pallas as pl
from jax.experimental.pallas import tpu as pltpu
from jax.experimental.pallas import tpu_sc as plsc

D = 64
V = 1000000
V2 = V // 2
NSEQ = 4096
T = 200
NW = 32
SB = 128
L = 16
DU = 16

_mesh = plsc.VectorSubcoreMesh(core_axis_name="c", subcore_axis_name="s")


@functools.partial(
    pl.kernel,
    mesh=_mesh,
    out_type=jax.ShapeDtypeStruct((T, D, NSEQ), jnp.float32),
    scratch_types=[
        pltpu.VMEM((T, SB), jnp.int32),
        pltpu.VMEM((8, SB), jnp.int32),
        pltpu.VMEM((SB, 2 * D), jnp.float32),
        pltpu.VMEM((SB, 2 * D), jnp.float32),
        pltpu.VMEM((D, SB), jnp.float32),
        pltpu.VMEM((D, SB), jnp.float32),
        pltpu.SemaphoreType.DMA,
        pltpu.SemaphoreType.DMA,
    ],
    compiler_params=pltpu.CompilerParams(use_tc_tiling_on_sc=True,
                                         needs_layout_passes=False),
)
def _gather_kernel(w2_hbm, idx_hbm, out_hbm, idx_v, q_v, rows0, rows1,
                   tr0, tr1, gsem0, gsem1):
    wid = lax.axis_index("s") * 2 + lax.axis_index("c")
    s0 = wid * SB

    pltpu.sync_copy(idx_hbm.at[:, pl.ds(s0, SB)], idx_v)

    def fire(t, b, row_buf, sem):
        for c in range(SB // L):
            q_v[b, pl.ds(c * L, L)] = lax.shift_right_logical(
                idx_v[t, pl.ds(c * L, L)], 1)
        pltpu.async_copy(w2_hbm.at[q_v.at[b]], row_buf, sem)

    def drain(row_buf, sem):
        pltpu.make_async_copy(w2_hbm.at[pl.ds(0, SB)], row_buf, sem).wait()

    def transpose(t, rows_k, tr_k):
        ridxs = [lax.iota(jnp.int32, L) + sb * L for sb in range(SB // L)]
        pars = [(idx_v[t, pl.ds(sb * L, L)] & 1) * D for sb in range(SB // L)]

        def dbody(j, _):
            d0 = j * DU
            for du in range(DU):
                d = d0 + du
                for sb in range(SB // L):
                    v = plsc.load_gather(rows_k, [ridxs[sb], pars[sb] + d])
                    tr_k[d, pl.ds(sb * L, L)] = v
            return 0

        lax.fori_loop(0, D // DU, dbody, 0)

    def writeback(tr_buf, t):
        pltpu.sync_copy(tr_buf, out_hbm.at[t, :, pl.ds(s0, SB)])

    fire(0, 0, rows0, gsem0)

    def body(h, _):
        a = 2 * h
        fire(a + 1, 1, rows1, gsem1)
        drain(rows0, gsem0)
        transpose(a, rows0, tr0)
        writeback(tr0, a)

        @pl.when(a + 2 < T)
        def _():
            fire(a + 2, 0, rows0, gsem0)

        drain(rows1, gsem1)
        transpose(a + 1, rows1, tr1)
        writeback(tr1, a + 1)
        return 0

    lax.fori_loop(0, T // 2, body, 0)


def kernel(idx, embed_weight):
    idx_t = idx.astype(jnp.int32).T
    w2 = embed_weight.reshape(V2, 2 * D)
    out = _gather_kernel(w2, idx_t)
    return out.transpose(2, 0, 1)


# speedup vs baseline: 1.5402x; 1.5402x over previous
import functools

import jax
import jax.numpy as jnp
from jax import lax
from jax.experimental import pallas as pl
from jax.experimental.pallas import tpu as pltpu
from jax.experimental.pallas import tpu_sc as plsc

D = 64
NSEQ = 4096
T = 200
SEGS = ((0, 104), (104, 96))
NW = 32
SEQ_PER_W = NSEQ // NW
S = 4
N_CHUNKS = SEQ_PER_W // S

_mesh = plsc.VectorSubcoreMesh(core_axis_name="c", subcore_axis_name="s")


@functools.partial(
    pl.kernel,
    mesh=_mesh,
    out_type=jax.ShapeDtypeStruct((NSEQ, T, D), jnp.float32),
    scratch_types=[
        pltpu.VMEM((S, T), jnp.int32),
        pltpu.VMEM((S, T), jnp.int32),
        pltpu.VMEM((S, T, D), jnp.float32),
        pltpu.VMEM((S, T, D), jnp.float32),
        pltpu.SemaphoreType.DMA,
        pltpu.SemaphoreType.DMA,
    ],
    compiler_params=pltpu.CompilerParams(use_tc_tiling_on_sc=False),
)
def _gather_kernel(table_hbm, idx_hbm, out_hbm, idx0, idx1, rows0, rows1,
                   gsem0, gsem1):
    wid = lax.axis_index("s") * 2 + lax.axis_index("c")
    seq_base = wid * SEQ_PER_W

    def fire(i, idx_buf, row_buf, sem):
        pltpu.sync_copy(idx_hbm.at[pl.ds(seq_base + i * S, S)], idx_buf)
        for s in range(S):
            for off, length in SEGS:
                pltpu.async_copy(
                    table_hbm.at[idx_buf.at[s, pl.ds(off, length)]],
                    row_buf.at[s, pl.ds(off, length)],
                    sem,
                )

    def drain(row_buf, sem):
        pltpu.make_async_copy(out_hbm.at[pl.ds(0, S)], row_buf, sem).wait()

    def writeback(row_buf, i):
        pltpu.sync_copy(row_buf, out_hbm.at[pl.ds(seq_base + i * S, S)])

    fire(0, idx0, rows0, gsem0)

    def body(t, _):
        a = 2 * t
        fire(a + 1, idx1, rows1, gsem1)
        drain(rows0, gsem0)
        writeback(rows0, a)

        @pl.when(a + 2 < N_CHUNKS)
        def _():
            fire(a + 2, idx0, rows0, gsem0)

        drain(rows1, gsem1)
        writeback(rows1, a + 1)
        return 0

    lax.fori_loop(0, N_CHUNKS // 2, body, 0)


def kernel(idx, embed_weight):
    return _gather_kernel(embed_weight, idx.astype(jnp.int32))
